# baseline (speedup 1.0000x reference)
import functools

import jax
import jax.numpy as jnp
from jax import lax
from jax.experimental import pallas as pl
from jax.experimental.pallas import tpu as pltpu
from jax.experimental.pallas import tpu_sc as plsc

DIM = 1024
HEADS = 16
DIM_HEAD = 64
BUCKET_SIZE = 64
N_HASHES = 8
TOKEN_SELF_ATTN_VALUE = -50000.0


def _attn_body(sqkv_ref, sqkv_pb_ref, st_ref, st_pb_ref, so_ref):
    cur = sqkv_ref[0]
    pb = sqkv_pb_ref[0]
    st = st_ref[0]
    st_pb = st_pb_ref[0]

    C = cur.shape[0]
    prev = jnp.concatenate([pb[C - 1:C], cur[:C - 1]], axis=0)
    st_prev = jnp.concatenate([st_pb[C - 1:C], st[:C - 1]], axis=0)

    bq = cur[:, :, :DIM_HEAD]
    sv = cur[:, :, DIM_HEAD:]
    bqk_prev = prev[:, :, :DIM_HEAD]
    sv_prev = prev[:, :, DIM_HEAD:]

    def norm(z):
        n = jnp.sqrt(jnp.sum(z * z, axis=-1, keepdims=True))
        return z / jnp.maximum(n, 1e-12)

    bk = jnp.concatenate([norm(bq), norm(bqk_prev)], axis=1)
    bv = jnp.concatenate([sv, sv_prev], axis=1)
    bkv_t = jnp.concatenate([st, st_prev], axis=1)

    dots = jax.lax.dot_general(
        bq, bk, (((2,), (2,)), ((0,), (0,))),
        preferred_element_type=jnp.float32) * (DIM_HEAD ** -0.5)
    self_mask = st[:, :, None] == bkv_t[:, None, :]
    dots = jnp.where(self_mask, TOKEN_SELF_ATTN_VALUE, dots)
    m = jnp.max(dots, axis=-1, keepdims=True)
    s = jnp.sum(jnp.exp(dots - m), axis=-1, keepdims=True)
    lse = m + jnp.log(s)
    p = jnp.exp(dots - lse)
    bo = jax.lax.dot_general(
        p, bv, (((2,), (1,)), ((0,), (0,))),
        preferred_element_type=jnp.float32)
    so_ref[0] = jnp.concatenate(
        [bo, jnp.broadcast_to(lse, bo.shape)], axis=-1)


def _chunk_attention(sqkv, st):
    bh, nc, cs, d2 = sqkv.shape
    C = 8
    nb = nc // C
    grid = (bh, nb)
    blk4 = pl.BlockSpec((1, C, cs, d2), lambda i, j: (i, j, 0, 0))
    blk4p = pl.BlockSpec((1, C, cs, d2), lambda i, j: (i, (j + nb - 1) % nb, 0, 0))
    blk3 = pl.BlockSpec((1, C, cs), lambda i, j: (i, j, 0))
    blk3p = pl.BlockSpec((1, C, cs), lambda i, j: (i, (j + nb - 1) % nb, 0))
    so = pl.pallas_call(
        _attn_body,
        grid=grid,
        in_specs=[blk4, blk4p, blk3, blk3p],
        out_specs=blk4,
        out_shape=jax.ShapeDtypeStruct((bh, nc, cs, d2), jnp.float32),
    )(sqkv, sqkv, st, st)
    return so


def _sc_gather(table, gidx):
    R, D = table.shape
    N = gidx.shape[0]
    NW = 32
    CH = 128
    per_w = N // NW
    n_iter = per_w // CH
    mesh = plsc.VectorSubcoreMesh(core_axis_name="c", subcore_axis_name="s")

    @functools.partial(
        pl.kernel, mesh=mesh,
        out_type=jax.ShapeDtypeStruct((N, D), jnp.float32),
        scratch_types=[
            pltpu.VMEM((2, CH), jnp.int32),
            pltpu.VMEM((2, CH, D), jnp.float32),
            pltpu.SemaphoreType.DMA,
            pltpu.SemaphoreType.DMA,
        ],
    )
    def k(t_hbm, idx_hbm, o_hbm, idx_v, rows_v, sem_i, sem_r):
        wid = lax.axis_index("s") * 2 + lax.axis_index("c")
        base_w = wid * per_w

        def body(i, carry):
            slot = lax.rem(i, 2)
            base = base_w + i * CH
            pltpu.sync_copy(idx_hbm.at[pl.ds(base, CH)], idx_v.at[slot])
            pltpu.async_copy(t_hbm.at[idx_v.at[slot]], rows_v.at[slot],
                             sem_r).wait()
            pltpu.sync_copy(rows_v.at[slot], o_hbm.at[pl.ds(base, CH)])
            return carry

        lax.fori_loop(0, n_iter, body, 0)

    return k(table, gidx)


def _proj_hash_body(x_ref, w_ref, rot_ref, qkv_ref, bkt_ref):
    xb = x_ref[...]
    w = w_ref[...]
    qkv = jnp.dot(xb, w, preferred_element_type=jnp.float32)
    rot = rot_ref[...]
    iota64 = lax.broadcasted_iota(jnp.int32, (1, 64), 1)
    for h in range(HEADS):
        qh = qkv[:, h * DIM_HEAD:(h + 1) * DIM_HEAD]
        vh = qkv[:, DIM + h * DIM_HEAD:DIM + (h + 1) * DIM_HEAD]
        qkv_ref[h, :, :DIM_HEAD] = qh
        qkv_ref[h, :, DIM_HEAD:] = vh
        r = jnp.dot(qh, rot, preferred_element_type=jnp.float32)
        for hh in range(N_HASHES):
            rr = r[:, hh * 32:(hh + 1) * 32]
            c = jnp.concatenate([rr, -rr], axis=1)
            m = jnp.max(c, axis=1, keepdims=True)
            idx = jnp.min(jnp.where(c == m, iota64, 64), axis=1)
            bkt_ref[h, hh, :] = idx.astype(jnp.int32)


def _proj_hash(x2d, Wqkv, rot):
    T = x2d.shape[0]
    Tb = 512
    grid = (T // Tb,)
    qkv, bkt = pl.pallas_call(
        _proj_hash_body,
        grid=grid,
        in_specs=[
            pl.BlockSpec((Tb, DIM), lambda j: (j, 0)),
            pl.BlockSpec((DIM, 2 * DIM), lambda j: (0, 0)),
            pl.BlockSpec((DIM_HEAD, 256), lambda j: (0, 0)),
        ],
        out_specs=[
            pl.BlockSpec((HEADS, Tb, 2 * DIM_HEAD), lambda j: (0, j, 0)),
            pl.BlockSpec((HEADS, N_HASHES, Tb), lambda j: (0, 0, j)),
        ],
        out_shape=[
            jax.ShapeDtypeStruct((HEADS, T, 2 * DIM_HEAD), jnp.float32),
            jax.ShapeDtypeStruct((HEADS, N_HASHES, T), jnp.int32),
        ],
    )(x2d, Wqkv, rot)
    return qkv, bkt


def _combine_body(og_ref, wout_ref, bout_ref, y_ref):
    og = og_ref[...]
    o = og[:, :, :, :DIM_HEAD]
    l = og[:, :, :, DIM_HEAD:DIM_HEAD + 1]
    m = jnp.max(l, axis=1, keepdims=True)
    se = jnp.sum(jnp.exp(l - m), axis=1, keepdims=True)
    lse = m + jnp.log(se)
    probs = jnp.exp(l - lse)
    ow = jnp.sum(o * probs, axis=1)
    acc = jnp.concatenate([ow[h] for h in range(HEADS)], axis=1)
    y_ref[...] = (jnp.dot(acc, wout_ref[...],
                          preferred_element_type=jnp.float32)
                  + bout_ref[...])


def _combine_outproj(og, Wout, bout):
    _, _, T, _ = og.shape
    Tb = 256
    grid = (T // Tb,)
    y = pl.pallas_call(
        _combine_body,
        grid=grid,
        in_specs=[
            pl.BlockSpec((HEADS, N_HASHES, Tb, 2 * DIM_HEAD),
                         lambda j: (0, 0, j, 0)),
            pl.BlockSpec((DIM, DIM), lambda j: (0, 0)),
            pl.BlockSpec((1, DIM), lambda j: (0, 0)),
        ],
        out_specs=pl.BlockSpec((Tb, DIM), lambda j: (j, 0)),
        out_shape=jax.ShapeDtypeStruct((T, DIM), jnp.float32),
    )(og, Wout, bout.reshape(1, DIM))
    return y


def _sc_sort(buckets_flat):
    N = buckets_flat.shape[0]
    S = 4096
    NSEG = N // S
    NW = 32
    segs_per_w = NSEG // NW
    mesh = plsc.VectorSubcoreMesh(core_axis_name="c", subcore_axis_name="s")

    @functools.partial(
        pl.kernel, mesh=mesh,
        out_type=[
            jax.ShapeDtypeStruct((N,), jnp.int32),
            jax.ShapeDtypeStruct((N,), jnp.int32),
        ],
        compiler_params=pltpu.CompilerParams(needs_layout_passes=False),
        scratch_types=[
            pltpu.VMEM((16 + S,), jnp.int32),
            pltpu.VMEM((S,), jnp.int32),
            pltpu.VMEM((S,), jnp.int32),
            pltpu.VMEM((64,), jnp.int32),
        ],
    )
    def k(b_hbm, st_hbm, undo_hbm, bk_v, st_v, undo_v, offs_v):
        wid = lax.axis_index("s") * 2 + lax.axis_index("c")
        lanes = lax.iota(jnp.int32, 16)
        ones = jnp.ones((16,), jnp.int32)

        def do_segment(si, carry):
            seg = wid * segs_per_w + si
            h = lax.rem(seg, 8)
            base = seg * S
            bk_v[pl.ds(0, 16)] = jnp.full((16,), -1, jnp.int32)
            pltpu.sync_copy(b_hbm.at[pl.ds(base, S)], bk_v.at[pl.ds(16, S)])

            offs_v[pl.ds(0, 16)] = jnp.zeros((16,), jnp.int32)
            offs_v[pl.ds(16, 16)] = jnp.zeros((16,), jnp.int32)
            offs_v[pl.ds(32, 16)] = jnp.zeros((16,), jnp.int32)
            offs_v[pl.ds(48, 16)] = jnp.zeros((16,), jnp.int32)

            def hist(i, c):
                kv = bk_v[pl.ds(16 + i * 16, 16)]
                plsc.addupdate_scatter(offs_v, [kv], ones)
                return c

            lax.fori_loop(0, S // 16, hist, 0)

            c0 = offs_v[pl.ds(0, 16)]
            c1 = offs_v[pl.ds(16, 16)]
            c2 = offs_v[pl.ds(32, 16)]
            c3 = offs_v[pl.ds(48, 16)]
            i0 = plsc.cumsum(c0)
            i1 = plsc.cumsum(c1)
            i2 = plsc.cumsum(c2)
            i3 = plsc.cumsum(c3)
            t0 = jnp.sum(c0)
            t1 = t0 + jnp.sum(c1)
            t2 = t1 + jnp.sum(c2)
            offs_v[pl.ds(0, 16)] = i0 - c0
            offs_v[pl.ds(16, 16)] = i1 - c1 + t0
            offs_v[pl.ds(32, 16)] = i2 - c2 + t1
            offs_v[pl.ds(48, 16)] = i3 - c3 + t2

            zeros16 = jnp.zeros((16,), jnp.int32)

            def place(i, c):
                kv = bk_v[pl.ds(16 + i * 16, 16)]
                eqb = zeros16
                for j in range(1, 16):
                    win = bk_v[pl.ds(16 + i * 16 - j, 16)]
                    eqb = eqb + jnp.where((kv == win) & (lanes >= j),
                                          ones, zeros16)
                pos = plsc.load_gather(offs_v, [kv]) + eqb
                plsc.addupdate_scatter(offs_v, [kv], ones)
                tvec = i * 16 + lanes
                plsc.store_scatter(st_v, [pos], tvec)
                undo_v[pl.ds(i * 16, 16)] = pos + h * S
                return c

            lax.fori_loop(0, S // 16, place, 0)

            pltpu.sync_copy(st_v, st_hbm.at[pl.ds(base, S)])
            pltpu.sync_copy(undo_v, undo_hbm.at[pl.ds(base, S)])
            return carry

        lax.fori_loop(0, segs_per_w, do_segment, 0)

    return k(buckets_flat)


def _logsumexp(x, axis):
    m = jnp.max(x, axis=axis, keepdims=True)
    return m + jnp.log(jnp.sum(jnp.exp(x - m), axis=axis, keepdims=True))


def kernel(x, Wqk, Wv, Wout, bout):
    b, t, e = x.shape
    seqlen = t
    bh = b * HEADS
    n_buckets = seqlen // BUCKET_SIZE

    rot = jax.random.normal(jax.random.key(1),
                            (1, DIM_HEAD, N_HASHES, n_buckets // 2),
                            dtype=jnp.float32)
    rot2d = rot[0].reshape(DIM_HEAD, N_HASHES * (n_buckets // 2))

    Wqkv = jnp.concatenate([Wqk, Wv], axis=1)
    qkv_table, buckets = _proj_hash(x.reshape(t, e), Wqkv, rot2d)

    st_flat, undo_flat = _sc_sort(buckets.reshape(-1))
    st = st_flat.reshape(bh, N_HASHES * seqlen)
    undo_sort = undo_flat.reshape(bh, N_HASHES * seqlen)

    gidx = (jnp.arange(bh, dtype=jnp.int32)[:, None] * seqlen + st).reshape(-1)
    sqkv = _sc_gather(qkv_table.reshape(bh * seqlen, 2 * DIM_HEAD), gidx)
    n_chunks = N_HASHES * n_buckets
    sqkv = sqkv.reshape(bh, n_chunks, BUCKET_SIZE, 2 * DIM_HEAD)
    stc = st.reshape(bh, n_chunks, BUCKET_SIZE)

    so = _chunk_attention(sqkv, stc)

    n_sorted = N_HASHES * seqlen
    uidx = (jnp.arange(bh, dtype=jnp.int32)[:, None] * n_sorted
            + undo_sort).reshape(-1)
    og = _sc_gather(so.reshape(bh * n_sorted, 2 * DIM_HEAD), uidx)
    og = og.reshape(bh, N_HASHES, seqlen, 2 * DIM_HEAD)

    y = _combine_outproj(og, Wout, bout)
    return y.reshape(b, t, e)


# speedup vs baseline: 1.0560x; 1.0560x over previous
import functools

import jax
import jax.numpy as jnp
from jax import lax
from jax.experimental import pallas as pl
from jax.experimental.pallas import tpu as pltpu
from jax.experimental.pallas import tpu_sc as plsc

DIM = 1024
HEADS = 16
DIM_HEAD = 64
BUCKET_SIZE = 64
N_HASHES = 8
TOKEN_SELF_ATTN_VALUE = -50000.0


def _attn_body(sqkv_ref, sqkv_pb_ref, st_ref, st_pb_ref, so_ref):
    cur = sqkv_ref[0]
    pb = sqkv_pb_ref[0]
    st = st_ref[0]
    st_pb = st_pb_ref[0]

    C = cur.shape[0]
    prev = jnp.concatenate([pb[C - 1:C], cur[:C - 1]], axis=0)
    st_prev = jnp.concatenate([st_pb[C - 1:C], st[:C - 1]], axis=0)

    bq = cur[:, :, :DIM_HEAD]
    sv = cur[:, :, DIM_HEAD:]
    bqk_prev = prev[:, :, :DIM_HEAD]
    sv_prev = prev[:, :, DIM_HEAD:]

    def norm(z):
        n = jnp.sqrt(jnp.sum(z * z, axis=-1, keepdims=True))
        return z / jnp.maximum(n, 1e-12)

    bk = jnp.concatenate([norm(bq), norm(bqk_prev)], axis=1)
    bv = jnp.concatenate([sv, sv_prev], axis=1)
    bkv_t = jnp.concatenate([st, st_prev], axis=1)

    dots = jax.lax.dot_general(
        bq, bk, (((2,), (2,)), ((0,), (0,))),
        preferred_element_type=jnp.float32) * (DIM_HEAD ** -0.5)
    self_mask = st[:, :, None] == bkv_t[:, None, :]
    dots = jnp.where(self_mask, TOKEN_SELF_ATTN_VALUE, dots)
    m = jnp.max(dots, axis=-1, keepdims=True)
    s = jnp.sum(jnp.exp(dots - m), axis=-1, keepdims=True)
    lse = m + jnp.log(s)
    p = jnp.exp(dots - lse)
    bo = jax.lax.dot_general(
        p, bv, (((2,), (1,)), ((0,), (0,))),
        preferred_element_type=jnp.float32)
    so_ref[0] = jnp.concatenate(
        [bo, jnp.broadcast_to(lse, bo.shape)], axis=-1)


def _chunk_attention(sqkv, st):
    bh, nc, cs, d2 = sqkv.shape
    C = 8
    nb = nc // C
    grid = (bh, nb)
    blk4 = pl.BlockSpec((1, C, cs, d2), lambda i, j: (i, j, 0, 0))
    blk4p = pl.BlockSpec((1, C, cs, d2), lambda i, j: (i, (j + nb - 1) % nb, 0, 0))
    blk3 = pl.BlockSpec((1, C, cs), lambda i, j: (i, j, 0))
    blk3p = pl.BlockSpec((1, C, cs), lambda i, j: (i, (j + nb - 1) % nb, 0))
    so = pl.pallas_call(
        _attn_body,
        grid=grid,
        in_specs=[blk4, blk4p, blk3, blk3p],
        out_specs=blk4,
        out_shape=jax.ShapeDtypeStruct((bh, nc, cs, d2), jnp.float32),
    )(sqkv, sqkv, st, st)
    return so


def _sc_gather(table, gidx):
    R, D = table.shape
    N = gidx.shape[0]
    NW = 32
    CH = 128
    per_w = N // NW
    n_iter = per_w // CH
    mesh = plsc.VectorSubcoreMesh(core_axis_name="c", subcore_axis_name="s")

    @functools.partial(
        pl.kernel, mesh=mesh,
        out_type=jax.ShapeDtypeStruct((N, D), jnp.float32),
        scratch_types=[
            pltpu.VMEM((2, CH), jnp.int32),
            pltpu.VMEM((2, CH, D), jnp.float32),
            pltpu.SemaphoreType.DMA,
            pltpu.SemaphoreType.DMA,
        ],
    )
    def k(t_hbm, idx_hbm, o_hbm, idx_v, rows_v, sem_i, sem_r):
        wid = lax.axis_index("s") * 2 + lax.axis_index("c")
        base_w = wid * per_w

        def body(i, carry):
            slot = lax.rem(i, 2)
            base = base_w + i * CH
            pltpu.sync_copy(idx_hbm.at[pl.ds(base, CH)], idx_v.at[slot])
            pltpu.async_copy(t_hbm.at[idx_v.at[slot]], rows_v.at[slot],
                             sem_r).wait()
            pltpu.sync_copy(rows_v.at[slot], o_hbm.at[pl.ds(base, CH)])
            return carry

        lax.fori_loop(0, n_iter, body, 0)

    return k(table, gidx)


def _proj_hash_body(x_ref, w_ref, rot_ref, qkv_ref, bkt_ref):
    xb = x_ref[...]
    w = w_ref[...]
    qkv = jnp.dot(xb, w, preferred_element_type=jnp.float32)
    rot = rot_ref[...]
    Tb = xb.shape[0]
    iota64 = lax.broadcasted_iota(jnp.int32, (1, 1, 64), 2)
    for h in range(HEADS):
        qh = qkv[:, h * DIM_HEAD:(h + 1) * DIM_HEAD]
        vh = qkv[:, DIM + h * DIM_HEAD:DIM + (h + 1) * DIM_HEAD]
        qkv_ref[h, :, :DIM_HEAD] = qh
        qkv_ref[h, :, DIM_HEAD:] = vh
        r = jnp.dot(qh, rot, preferred_element_type=jnp.float32)
        r3 = r.reshape(Tb, N_HASHES, 32)
        c = jnp.concatenate([r3, -r3], axis=-1)
        m = jnp.max(c, axis=-1, keepdims=True)
        idx = jnp.min(jnp.where(c == m, iota64, 64), axis=-1)
        bkt_ref[h] = idx.astype(jnp.int32)


def _proj_hash(x2d, Wqkv, rot):
    T = x2d.shape[0]
    Tb = 512
    grid = (T // Tb,)
    qkv, bkt = pl.pallas_call(
        _proj_hash_body,
        grid=grid,
        in_specs=[
            pl.BlockSpec((Tb, DIM), lambda j: (j, 0)),
            pl.BlockSpec((DIM, 2 * DIM), lambda j: (0, 0)),
            pl.BlockSpec((DIM_HEAD, 256), lambda j: (0, 0)),
        ],
        out_specs=[
            pl.BlockSpec((HEADS, Tb, 2 * DIM_HEAD), lambda j: (0, j, 0)),
            pl.BlockSpec((HEADS, Tb, N_HASHES), lambda j: (0, j, 0)),
        ],
        out_shape=[
            jax.ShapeDtypeStruct((HEADS, T, 2 * DIM_HEAD), jnp.float32),
            jax.ShapeDtypeStruct((HEADS, T, N_HASHES), jnp.int32),
        ],
    )(x2d, Wqkv, rot)
    return qkv, bkt.transpose(0, 2, 1)


def _combine_body(og_ref, wout_ref, bout_ref, y_ref):
    og = og_ref[...]
    o = og[:, :, :, :DIM_HEAD]
    l = og[:, :, :, DIM_HEAD:]
    m = jnp.max(l, axis=1, keepdims=True)
    se = jnp.sum(jnp.exp(l - m), axis=1, keepdims=True)
    lse = m + jnp.log(se)
    probs = jnp.exp(l - lse)
    ow = jnp.sum(o * probs, axis=1)
    acc = jnp.concatenate([ow[h] for h in range(HEADS)], axis=1)
    y_ref[...] = (jnp.dot(acc, wout_ref[...],
                          preferred_element_type=jnp.float32)
                  + bout_ref[...])


def _combine_outproj(og, Wout, bout):
    _, _, T, _ = og.shape
    Tb = 256
    grid = (T // Tb,)
    y = pl.pallas_call(
        _combine_body,
        grid=grid,
        in_specs=[
            pl.BlockSpec((HEADS, N_HASHES, Tb, 2 * DIM_HEAD),
                         lambda j: (0, 0, j, 0)),
            pl.BlockSpec((DIM, DIM), lambda j: (0, 0)),
            pl.BlockSpec((1, DIM), lambda j: (0, 0)),
        ],
        out_specs=pl.BlockSpec((Tb, DIM), lambda j: (j, 0)),
        out_shape=jax.ShapeDtypeStruct((T, DIM), jnp.float32),
    )(og, Wout, bout.reshape(1, DIM))
    return y


def _sc_sort(buckets_flat):
    N = buckets_flat.shape[0]
    S = 4096
    NSEG = N // S
    NW = 32
    segs_per_w = NSEG // NW
    mesh = plsc.VectorSubcoreMesh(core_axis_name="c", subcore_axis_name="s")

    @functools.partial(
        pl.kernel, mesh=mesh,
        out_type=[
            jax.ShapeDtypeStruct((N,), jnp.int32),
            jax.ShapeDtypeStruct((N,), jnp.int32),
        ],
        compiler_params=pltpu.CompilerParams(needs_layout_passes=False),
        scratch_types=[
            pltpu.VMEM((16 + S,), jnp.int32),
            pltpu.VMEM((S,), jnp.int32),
            pltpu.VMEM((S,), jnp.int32),
            pltpu.VMEM((64,), jnp.int32),
        ],
    )
    def k(b_hbm, st_hbm, undo_hbm, bk_v, st_v, undo_v, offs_v):
        wid = lax.axis_index("s") * 2 + lax.axis_index("c")
        lanes = lax.iota(jnp.int32, 16)
        ones = jnp.ones((16,), jnp.int32)

        def do_segment(si, carry):
            seg = wid * segs_per_w + si
            h = lax.rem(seg, 8)
            base = seg * S
            bk_v[pl.ds(0, 16)] = jnp.full((16,), -1, jnp.int32)
            pltpu.sync_copy(b_hbm.at[pl.ds(base, S)], bk_v.at[pl.ds(16, S)])

            offs_v[pl.ds(0, 16)] = jnp.zeros((16,), jnp.int32)
            offs_v[pl.ds(16, 16)] = jnp.zeros((16,), jnp.int32)
            offs_v[pl.ds(32, 16)] = jnp.zeros((16,), jnp.int32)
            offs_v[pl.ds(48, 16)] = jnp.zeros((16,), jnp.int32)

            def hist(i, c):
                kv = bk_v[pl.ds(16 + i * 16, 16)]
                plsc.addupdate_scatter(offs_v, [kv], ones)
                return c

            lax.fori_loop(0, S // 16, hist, 0)

            c0 = offs_v[pl.ds(0, 16)]
            c1 = offs_v[pl.ds(16, 16)]
            c2 = offs_v[pl.ds(32, 16)]
            c3 = offs_v[pl.ds(48, 16)]
            i0 = plsc.cumsum(c0)
            i1 = plsc.cumsum(c1)
            i2 = plsc.cumsum(c2)
            i3 = plsc.cumsum(c3)
            t0 = jnp.sum(c0)
            t1 = t0 + jnp.sum(c1)
            t2 = t1 + jnp.sum(c2)
            offs_v[pl.ds(0, 16)] = i0 - c0
            offs_v[pl.ds(16, 16)] = i1 - c1 + t0
            offs_v[pl.ds(32, 16)] = i2 - c2 + t1
            offs_v[pl.ds(48, 16)] = i3 - c3 + t2

            zeros16 = jnp.zeros((16,), jnp.int32)

            def place(i, c):
                kv = bk_v[pl.ds(16 + i * 16, 16)]
                eqb = zeros16
                for j in range(1, 16):
                    win = bk_v[pl.ds(16 + i * 16 - j, 16)]
                    eqb = eqb + jnp.where((kv == win) & (lanes >= j),
                                          ones, zeros16)
                pos = plsc.load_gather(offs_v, [kv]) + eqb
                plsc.addupdate_scatter(offs_v, [kv], ones)
                tvec = i * 16 + lanes
                plsc.store_scatter(st_v, [pos], tvec)
                undo_v[pl.ds(i * 16, 16)] = pos + h * S
                return c

            lax.fori_loop(0, S // 16, place, 0)

            pltpu.sync_copy(st_v, st_hbm.at[pl.ds(base, S)])
            pltpu.sync_copy(undo_v, undo_hbm.at[pl.ds(base, S)])
            return carry

        lax.fori_loop(0, segs_per_w, do_segment, 0)

    return k(buckets_flat)


def _logsumexp(x, axis):
    m = jnp.max(x, axis=axis, keepdims=True)
    return m + jnp.log(jnp.sum(jnp.exp(x - m), axis=axis, keepdims=True))


def kernel(x, Wqk, Wv, Wout, bout):
    b, t, e = x.shape
    seqlen = t
    bh = b * HEADS
    n_buckets = seqlen // BUCKET_SIZE

    rot = jax.random.normal(jax.random.key(1),
                            (1, DIM_HEAD, N_HASHES, n_buckets // 2),
                            dtype=jnp.float32)
    rot2d = rot[0].reshape(DIM_HEAD, N_HASHES * (n_buckets // 2))

    Wqkv = jnp.concatenate([Wqk, Wv], axis=1)
    qkv_table, buckets = _proj_hash(x.reshape(t, e), Wqkv, rot2d)

    st_flat, undo_flat = _sc_sort(buckets.reshape(-1))
    st = st_flat.reshape(bh, N_HASHES * seqlen)
    undo_sort = undo_flat.reshape(bh, N_HASHES * seqlen)

    gidx = (jnp.arange(bh, dtype=jnp.int32)[:, None] * seqlen + st).reshape(-1)
    sqkv = _sc_gather(qkv_table.reshape(bh * seqlen, 2 * DIM_HEAD), gidx)
    n_chunks = N_HASHES * n_buckets
    sqkv = sqkv.reshape(bh, n_chunks, BUCKET_SIZE, 2 * DIM_HEAD)
    stc = st.reshape(bh, n_chunks, BUCKET_SIZE)

    so = _chunk_attention(sqkv, stc)

    n_sorted = N_HASHES * seqlen
    uidx = (jnp.arange(bh, dtype=jnp.int32)[:, None] * n_sorted
            + undo_sort).reshape(-1)
    og = _sc_gather(so.reshape(bh * n_sorted, 2 * DIM_HEAD), uidx)
    og = og.reshape(bh, N_HASHES, seqlen, 2 * DIM_HEAD)

    y = _combine_outproj(og, Wout, bout)
    return y.reshape(b, t, e)
